# baseline (speedup 1.0000x reference)
import functools

import jax
import jax.numpy as jnp
from jax import lax
from jax.experimental import pallas as pl
from jax.experimental.pallas import tpu as pltpu
from jax.experimental.pallas import tpu_sc as plsc

N_ITEMS = 100000
IN_DIM = 64
OUT_DIM = 64
N_COM = 5
TOP_K = 8
FEAT = N_COM * OUT_DIM
NPL = 3

NC = 2
NS = 16
NW = NC * NS

_Z_ROWS_BLK = 1024
_DN_T = (((0,), (0,)), ((), ()))


def _zbuild_body(ft_ref, w1a_ref, b1a_ref, w1b_ref, b1b_ref, w2_ref, b2_ref, z_ref):
    zero = jnp.zeros((_Z_ROWS_BLK, NPL * 128 - FEAT), jnp.float32)
    cols = []
    for c in range(N_COM):
        xa_t = ft_ref[2 * c].astype(jnp.bfloat16)
        xb_t = ft_ref[2 * c + 1].astype(jnp.bfloat16)
        ha = jnp.maximum(
            lax.dot_general(xa_t, w1a_ref[c].astype(jnp.bfloat16), _DN_T,
                            preferred_element_type=jnp.float32) + b1a_ref[c], 0.0)
        hb = jnp.maximum(
            lax.dot_general(xb_t, w1b_ref[c].astype(jnp.bfloat16), _DN_T,
                            preferred_element_type=jnp.float32) + b1b_ref[c], 0.0)
        cols.append(
            jnp.dot((ha + hb).astype(jnp.bfloat16),
                    w2_ref[c].astype(jnp.bfloat16),
                    preferred_element_type=jnp.float32) + b2_ref[c])
    z = jnp.concatenate(cols + [zero], axis=1)
    for j in range(NPL):
        z_ref[j] = z[:, j * 128:(j + 1) * 128]


def _build_z(ft_t, W1a, b1a, W1b, b1b, W2, b2):
    n = ft_t.shape[2]
    grid = (n + _Z_ROWS_BLK - 1) // _Z_ROWS_BLK
    return pl.pallas_call(
        _zbuild_body,
        grid=(grid,),
        in_specs=[
            pl.BlockSpec((2 * N_COM, IN_DIM, _Z_ROWS_BLK), lambda i: (0, 0, i)),
            pl.BlockSpec((N_COM, IN_DIM, OUT_DIM), lambda i: (0, 0, 0)),
            pl.BlockSpec((N_COM, OUT_DIM), lambda i: (0, 0)),
            pl.BlockSpec((N_COM, IN_DIM, OUT_DIM), lambda i: (0, 0, 0)),
            pl.BlockSpec((N_COM, OUT_DIM), lambda i: (0, 0)),
            pl.BlockSpec((N_COM, OUT_DIM, OUT_DIM), lambda i: (0, 0, 0)),
            pl.BlockSpec((N_COM, OUT_DIM), lambda i: (0, 0)),
        ],
        out_specs=pl.BlockSpec((NPL, _Z_ROWS_BLK, 128), lambda i: (0, i, 0)),
        out_shape=jax.ShapeDtypeStruct((NPL, n, 128), jnp.float32),
    )(ft_t, W1a, b1a, W1b, b1b, W2, b2)


def _resolve_kernel(mp, ic):
    cpw = mp // NW
    nic = cpw // ic
    mesh = plsc.VectorSubcoreMesh(core_axis_name="c", subcore_axis_name="s")

    @functools.partial(
        pl.kernel,
        out_type=jax.ShapeDtypeStruct((NW, TOP_K, cpw), jnp.int32),
        mesh=mesh,
        scratch_types=[
            pltpu.VMEM((TOP_K, cpw), jnp.int32),
            pltpu.VMEM((TOP_K, cpw), jnp.int32),
            pltpu.SemaphoreType.DMA,
        ],
        compiler_params=pltpu.CompilerParams(use_tc_tiling_on_sc=False),
    )
    def k(nbrflat_hbm, keys_hbm, out_hbm, keys_v, idx_v, sem):
        wid = lax.axis_index("s") * NC + lax.axis_index("c")
        pltpu.sync_copy(keys_hbm.at[wid], keys_v)
        cps = [
            pltpu.async_copy(
                nbrflat_hbm.at[keys_v.at[kk].at[pl.ds(j * ic, ic)]],
                idx_v.at[kk].at[pl.ds(j * ic, ic)],
                sem,
            )
            for kk in range(TOP_K)
            for j in range(nic)
        ]
        for c in cps:
            c.wait()
        pltpu.sync_copy(idx_v, out_hbm.at[wid])

    return k


def _bag_kernel(mp, acc_rows, gi):
    cpw = mp // NW
    ng = cpw // gi
    acc_rows_out = 2048
    stripe = acc_rows_out // NS
    mesh = plsc.VectorSubcoreMesh(core_axis_name="c", subcore_axis_name="s")

    scratch = [
        pltpu.VMEM((TOP_K, cpw), jnp.int32),
        pltpu.VMEM((2, NPL, gi, 128), jnp.float32),
        pltpu.VMEM((ng, gi), jnp.int32),
        pltpu.VMEM_SHARED((NPL, acc_rows, 128), jnp.float32),
        pltpu.SemaphoreType.DMA,
        pltpu.SemaphoreType.DMA,
    ]

    @functools.partial(
        pl.kernel,
        out_type=jax.ShapeDtypeStruct((NC, NPL, acc_rows_out, 128), jnp.float32),
        mesh=mesh,
        scratch_types=scratch,
        compiler_params=pltpu.CompilerParams(use_tc_tiling_on_sc=False),
    )
    def k(z_hbm, ridx_hbm, dst_hbm, zeros_hbm, out_hbm,
          idx_v, vec_v, dst_v, acc_sh, sem, sem2):
        cid = lax.axis_index("c")
        sid = lax.axis_index("s")
        wid = sid * NC + cid

        @pl.when(sid == 0)
        def _zero():
            for j in range(NPL):
                pltpu.sync_copy(zeros_hbm, acc_sh.at[j].at[pl.ds(0, acc_rows_out)])

        pltpu.sync_copy(ridx_hbm.at[wid], idx_v)
        pltpu.sync_copy(dst_hbm.at[wid], dst_v)
        plsc.subcore_barrier()

        sems = [sem, sem2]

        def issue_first(g, b):
            return [
                pltpu.async_copy(
                    z_hbm.at[j].at[idx_v.at[0].at[pl.ds(g * gi, gi)]],
                    vec_v.at[b].at[j], sems[b])
                for j in range(NPL)
            ]

        def issue_adds(g, b):
            return [
                pltpu.async_copy(
                    z_hbm.at[j].at[idx_v.at[kk].at[pl.ds(g * gi, gi)]],
                    vec_v.at[b].at[j], sems[b], add=True)
                for kk in range(1, TOP_K)
                for j in range(NPL)
            ]

        pend_first = issue_first(0, 0)
        for g in range(ng):
            b = g % 2
            for c in pend_first:
                c.wait()
            adds = issue_adds(g, b)
            if g + 1 < ng:
                pend_first = issue_first(g + 1, 1 - b)
            for c in adds:
                c.wait()
            for j in range(NPL):
                pltpu.sync_copy(vec_v.at[b].at[j],
                                acc_sh.at[j].at[dst_v.at[g]], add=True)
        plsc.subcore_barrier()
        for j in range(NPL):
            pltpu.sync_copy(
                acc_sh.at[j].at[pl.ds(sid * stripe, stripe)],
                out_hbm.at[cid].at[j].at[pl.ds(sid * stripe, stripe)],
            )

    return k


def _finish_body(p_ref, su_ref, wi_ref, bi_ref, o_ref):
    logits = bi_ref[0, 0]
    for j in range(NPL):
        acc = (p_ref[0, j].astype(jnp.float32)
               + p_ref[1, j].astype(jnp.float32))
        item = acc[:1024] * (1.0 / TOP_K)
        user = acc[1024:2048] * su_ref[...]
        logits = logits + jnp.dot(item * user, wi_ref[j],
                                  preferred_element_type=jnp.float32)
    o_ref[...] = jax.nn.sigmoid(logits)


def _finish(partial, scale_user, W_int3, b_int):
    return pl.pallas_call(
        _finish_body,
        in_specs=[
            pl.BlockSpec(partial.shape, lambda: (0, 0, 0, 0)),
            pl.BlockSpec((1024, 1), lambda: (0, 0)),
            pl.BlockSpec((NPL, 128, 1), lambda: (0, 0, 0)),
            pl.BlockSpec((1, 1), lambda: (0, 0)),
        ],
        out_specs=pl.BlockSpec((1024, 1), lambda: (0, 0)),
        out_shape=jax.ShapeDtypeStruct((1024, 1), jnp.float32),
    )(partial, scale_user, W_int3, b_int)


def kernel(item_idxs, user_items_flat, cu_seqlens, neighbor_table, feat_tables,
           W1a, b1a, W1b, b1b, W2, b2, W_int, b_int):
    B = item_idxs.shape[0]
    T = user_items_flat.shape[0]
    n_items = neighbor_table.shape[0]
    M = B + T
    gi = 80
    ic = 80
    cpw = ((M + NW * gi - 1) // (NW * gi)) * gi
    mp = cpw * NW
    acc_rows = 2 * B + 8

    item_idxs = item_idxs.astype(jnp.int32)
    user_items_flat = user_items_flat.astype(jnp.int32)
    cu_seqlens = cu_seqlens.astype(jnp.int32)
    neighbor_table = neighbor_table.astype(jnp.int32)

    ft_t = jnp.transpose(feat_tables, (0, 2, 1))
    z3 = _build_z(ft_t, W1a, b1a, W1b, b1b, W2, b2)

    all_idx = jnp.concatenate(
        [item_idxs, user_items_flat,
         jnp.zeros((mp - M,), jnp.int32)])
    seg_ids = jnp.cumsum(
        jnp.zeros((T,), jnp.int32).at[cu_seqlens[1:-1]].add(1))
    dst = jnp.concatenate(
        [jnp.arange(B, dtype=jnp.int32), B + seg_ids,
         jnp.full((mp - M,), 2 * B, jnp.int32)])
    dst3 = dst.reshape(NW, cpw // gi, gi)

    nbr_flat = jnp.transpose(neighbor_table).reshape(-1)
    keys = (all_idx.reshape(NW, 1, cpw)
            + (n_items * jnp.arange(TOP_K, dtype=jnp.int32)).reshape(1, TOP_K, 1))

    zeros = jnp.zeros((2048, 128), jnp.float32)
    ridx = _resolve_kernel(mp, ic)(nbr_flat, keys)
    partial = _bag_kernel(mp, acc_rows, gi=gi)(z3, ridx, dst3, zeros)

    counts = jnp.diff(cu_seqlens).astype(jnp.float32)
    scale_user = (1.0 / (TOP_K * jnp.maximum(counts, 1.0))).reshape(B, 1)
    W_int3 = jnp.concatenate(
        [W_int.reshape(FEAT, 1),
         jnp.zeros((NPL * 128 - FEAT, 1), jnp.float32)]).reshape(NPL, 128, 1)
    pred = _finish(partial, scale_user, W_int3, b_int.reshape(1, 1))
    return pred.reshape(-1)


# speedup vs baseline: 1.0503x; 1.0503x over previous
import functools

import jax
import jax.numpy as jnp
from jax import lax
from jax.experimental import pallas as pl
from jax.experimental.pallas import tpu as pltpu
from jax.experimental.pallas import tpu_sc as plsc

N_ITEMS = 100000
IN_DIM = 64
OUT_DIM = 64
N_COM = 5
TOP_K = 8
FEAT = N_COM * OUT_DIM
NPL = 3

NC = 2
NS = 16
NW = NC * NS

_Z_ROWS_BLK = 1024
_DN_T = (((0,), (0,)), ((), ()))


def _zbuild_body(ft_ref, w1a_ref, b1a_ref, w1b_ref, b1b_ref, w2_ref, b2_ref, z_ref):
    zero = jnp.zeros((_Z_ROWS_BLK, NPL * 128 - FEAT), jnp.float32)
    cols = []
    for c in range(N_COM):
        xa_t = ft_ref[2 * c].astype(jnp.bfloat16)
        xb_t = ft_ref[2 * c + 1].astype(jnp.bfloat16)
        ha = jnp.maximum(
            lax.dot_general(xa_t, w1a_ref[c].astype(jnp.bfloat16), _DN_T,
                            preferred_element_type=jnp.float32) + b1a_ref[c], 0.0)
        hb = jnp.maximum(
            lax.dot_general(xb_t, w1b_ref[c].astype(jnp.bfloat16), _DN_T,
                            preferred_element_type=jnp.float32) + b1b_ref[c], 0.0)
        cols.append(
            jnp.dot((ha + hb).astype(jnp.bfloat16),
                    w2_ref[c].astype(jnp.bfloat16),
                    preferred_element_type=jnp.float32) + b2_ref[c])
    z = jnp.concatenate(cols + [zero], axis=1)
    for j in range(NPL):
        z_ref[j] = z[:, j * 128:(j + 1) * 128]


def _build_z(ft_t, W1a, b1a, W1b, b1b, W2, b2):
    n = ft_t.shape[2]
    grid = (n + _Z_ROWS_BLK - 1) // _Z_ROWS_BLK
    return pl.pallas_call(
        _zbuild_body,
        grid=(grid,),
        in_specs=[
            pl.BlockSpec((2 * N_COM, IN_DIM, _Z_ROWS_BLK), lambda i: (0, 0, i)),
            pl.BlockSpec((N_COM, IN_DIM, OUT_DIM), lambda i: (0, 0, 0)),
            pl.BlockSpec((N_COM, OUT_DIM), lambda i: (0, 0)),
            pl.BlockSpec((N_COM, IN_DIM, OUT_DIM), lambda i: (0, 0, 0)),
            pl.BlockSpec((N_COM, OUT_DIM), lambda i: (0, 0)),
            pl.BlockSpec((N_COM, OUT_DIM, OUT_DIM), lambda i: (0, 0, 0)),
            pl.BlockSpec((N_COM, OUT_DIM), lambda i: (0, 0)),
        ],
        out_specs=pl.BlockSpec((NPL, _Z_ROWS_BLK, 128), lambda i: (0, i, 0)),
        out_shape=jax.ShapeDtypeStruct((NPL, n, 128), jnp.float32),
    )(ft_t, W1a, b1a, W1b, b1b, W2, b2)


def _bag_kernel(mp, acc_rows, gi, ic):
    cpw = mp // NW
    ng = cpw // gi
    nic = cpw // ic
    acc_rows_out = 2048
    stripe = acc_rows_out // NS
    mesh = plsc.VectorSubcoreMesh(core_axis_name="c", subcore_axis_name="s")

    scratch = [
        pltpu.VMEM((TOP_K, cpw), jnp.int32),
        pltpu.VMEM((TOP_K, cpw), jnp.int32),
        pltpu.VMEM((2, NPL, gi, 128), jnp.float32),
        pltpu.VMEM((ng, gi), jnp.int32),
        pltpu.VMEM_SHARED((NPL, acc_rows, 128), jnp.float32),
        pltpu.SemaphoreType.DMA,
        pltpu.SemaphoreType.DMA,
        pltpu.SemaphoreType.DMA,
        pltpu.SemaphoreType.DMA,
    ]

    @functools.partial(
        pl.kernel,
        out_type=jax.ShapeDtypeStruct((NC, NPL, acc_rows_out, 128), jnp.float32),
        mesh=mesh,
        scratch_types=scratch,
        compiler_params=pltpu.CompilerParams(use_tc_tiling_on_sc=False),
    )
    def k(z_hbm, nbrflat_hbm, keys_hbm, dst_hbm, zeros_hbm, out_hbm,
          keys_v, idx_v, vec_v, dst_v, acc_sh, sem, sem2, sem3, sem4):
        cid = lax.axis_index("c")
        sid = lax.axis_index("s")
        wid = sid * NC + cid

        @pl.when(sid == 0)
        def _zero():
            for j in range(NPL):
                pltpu.sync_copy(zeros_hbm, acc_sh.at[j].at[pl.ds(0, acc_rows_out)])

        pltpu.sync_copy(keys_hbm.at[wid], keys_v)
        pltpu.sync_copy(dst_hbm.at[wid], dst_v)

        cps = [
            pltpu.async_copy(
                nbrflat_hbm.at[keys_v.at[kk].at[pl.ds(j * ic, ic)]],
                idx_v.at[kk].at[pl.ds(j * ic, ic)],
                sem,
            )
            for kk in range(TOP_K)
            for j in range(nic)
        ]
        for c in cps:
            c.wait()
        plsc.subcore_barrier()

        sems = [sem, sem2]

        def issue_first(g, b):
            return [
                pltpu.async_copy(
                    z_hbm.at[j].at[idx_v.at[0].at[pl.ds(g * gi, gi)]],
                    vec_v.at[b].at[j], sems[b])
                for j in range(NPL)
            ]

        def issue_adds(g, b):
            return [
                pltpu.async_copy(
                    z_hbm.at[j].at[idx_v.at[kk].at[pl.ds(g * gi, gi)]],
                    vec_v.at[b].at[j], sems[b], add=True)
                for kk in range(1, TOP_K)
                for j in range(NPL)
            ]

        scat_sems = [sem3, sem4]
        pend_scat = {0: [], 1: []}
        pend_first = issue_first(0, 0)
        for g in range(ng):
            b = g % 2
            for c in pend_first:
                c.wait()
            adds = issue_adds(g, b)
            if g + 1 < ng:
                for c in pend_scat[1 - b]:
                    c.wait()
                pend_first = issue_first(g + 1, 1 - b)
            for c in adds:
                c.wait()
            pend_scat[b] = [
                pltpu.async_copy(vec_v.at[b].at[j],
                                 acc_sh.at[j].at[dst_v.at[g]],
                                 scat_sems[b], add=True)
                for j in range(NPL)
            ]
        for b in (0, 1):
            for c in pend_scat[b]:
                c.wait()
        plsc.subcore_barrier()
        for j in range(NPL):
            pltpu.sync_copy(
                acc_sh.at[j].at[pl.ds(sid * stripe, stripe)],
                out_hbm.at[cid].at[j].at[pl.ds(sid * stripe, stripe)],
            )

    return k


def _finish_body(p_ref, su_ref, wi_ref, bi_ref, o_ref):
    logits = bi_ref[0, 0]
    for j in range(NPL):
        acc = (p_ref[0, j].astype(jnp.float32)
               + p_ref[1, j].astype(jnp.float32))
        item = acc[:1024] * (1.0 / TOP_K)
        user = acc[1024:2048] * su_ref[...]
        logits = logits + jnp.dot(item * user, wi_ref[j],
                                  preferred_element_type=jnp.float32)
    o_ref[...] = jax.nn.sigmoid(logits)


def _finish(partial, scale_user, W_int3, b_int):
    return pl.pallas_call(
        _finish_body,
        in_specs=[
            pl.BlockSpec(partial.shape, lambda: (0, 0, 0, 0)),
            pl.BlockSpec((1024, 1), lambda: (0, 0)),
            pl.BlockSpec((NPL, 128, 1), lambda: (0, 0, 0)),
            pl.BlockSpec((1, 1), lambda: (0, 0)),
        ],
        out_specs=pl.BlockSpec((1024, 1), lambda: (0, 0)),
        out_shape=jax.ShapeDtypeStruct((1024, 1), jnp.float32),
    )(partial, scale_user, W_int3, b_int)


def kernel(item_idxs, user_items_flat, cu_seqlens, neighbor_table, feat_tables,
           W1a, b1a, W1b, b1b, W2, b2, W_int, b_int):
    B = item_idxs.shape[0]
    T = user_items_flat.shape[0]
    n_items = neighbor_table.shape[0]
    M = B + T
    gi = 80
    ic = 80
    cpw = ((M + NW * gi - 1) // (NW * gi)) * gi
    mp = cpw * NW
    acc_rows = 2 * B + 8

    item_idxs = item_idxs.astype(jnp.int32)
    user_items_flat = user_items_flat.astype(jnp.int32)
    cu_seqlens = cu_seqlens.astype(jnp.int32)
    neighbor_table = neighbor_table.astype(jnp.int32)

    ft_t = jnp.transpose(feat_tables, (0, 2, 1))
    z3 = _build_z(ft_t, W1a, b1a, W1b, b1b, W2, b2)

    all_idx = jnp.concatenate(
        [item_idxs, user_items_flat,
         jnp.zeros((mp - M,), jnp.int32)])
    seg_ids = jnp.cumsum(
        jnp.zeros((T,), jnp.int32).at[cu_seqlens[1:-1]].add(1))
    dst = jnp.concatenate(
        [jnp.arange(B, dtype=jnp.int32), B + seg_ids,
         jnp.full((mp - M,), 2 * B, jnp.int32)])
    dst3 = dst.reshape(NW, cpw // gi, gi)

    nbr_flat = jnp.transpose(neighbor_table).reshape(-1)
    keys = (all_idx.reshape(NW, 1, cpw)
            + (n_items * jnp.arange(TOP_K, dtype=jnp.int32)).reshape(1, TOP_K, 1))

    zeros = jnp.zeros((2048, 128), jnp.float32)
    partial = _bag_kernel(mp, acc_rows, gi=gi, ic=ic)(
        z3, nbr_flat, keys, dst3, zeros)

    counts = jnp.diff(cu_seqlens).astype(jnp.float32)
    scale_user = (1.0 / (TOP_K * jnp.maximum(counts, 1.0))).reshape(B, 1)
    W_int3 = jnp.concatenate(
        [W_int.reshape(FEAT, 1),
         jnp.zeros((NPL * 128 - FEAT, 1), jnp.float32)]).reshape(NPL, 128, 1)
    pred = _finish(partial, scale_user, W_int3, b_int.reshape(1, 1))
    return pred.reshape(-1)
